# initial kernel scaffold (unmeasured)
import jax
import jax.numpy as jnp
from jax import lax
from jax.experimental import pallas as pl
from jax.experimental.pallas import tpu as pltpu

N_DEV = 32


def kernel(x, w_mat):
    k_per = x.shape[1]
    m, n = x.shape[0], w_mat.shape[1]
    m_per = m // N_DEV

    def body(x_ref, w_ref, out_ref, partial_ref, acc_ref, send_sems, recv_sems):
        me = lax.axis_index("i")

        partial_ref[...] = jnp.dot(
            x_ref[...], w_ref[...], preferred_element_type=jnp.float32
        )

        rdmas = []
        for k in range(1, N_DEV):
            c = lax.rem(me + k, N_DEV)
            rdma = pltpu.make_async_remote_copy(
                src_ref=partial_ref.at[pl.ds(c * m_per, m_per), :],
                dst_ref=acc_ref.at[me],
                send_sem=send_sems.at[c],
                recv_sem=recv_sems.at[me],
                device_id=(c,),
                device_id_type=pl.DeviceIdType.MESH,
            )
            rdma.start()
            rdmas.append(rdma)

        acc_ref[me] = partial_ref[pl.ds(me * m_per, m_per), :]

        for k in range(1, N_DEV):
            j = lax.rem(me + k, N_DEV)
            recv = pltpu.make_async_remote_copy(
                src_ref=acc_ref.at[j],
                dst_ref=acc_ref.at[j],
                send_sem=send_sems.at[j],
                recv_sem=recv_sems.at[j],
                device_id=(j,),
                device_id_type=pl.DeviceIdType.MESH,
            )
            recv.wait_recv()

        for rdma in rdmas:
            rdma.wait_send()

        out_ref[...] = jnp.sum(acc_ref[...], axis=0)

    return pl.pallas_call(
        body,
        out_shape=jax.ShapeDtypeStruct((m_per, n), jnp.float32),
        in_specs=[
            pl.BlockSpec(memory_space=pltpu.VMEM),
            pl.BlockSpec(memory_space=pltpu.VMEM),
        ],
        out_specs=pl.BlockSpec(memory_space=pltpu.VMEM),
        scratch_shapes=[
            pltpu.VMEM((m, n), jnp.float32),
            pltpu.VMEM((N_DEV, m_per, n), jnp.float32),
            pltpu.SemaphoreType.DMA((N_DEV,)),
            pltpu.SemaphoreType.DMA((N_DEV,)),
        ],
        compiler_params=pltpu.CompilerParams(collective_id=0),
    )(x, w_mat)


# baseline (device time: 65288 ns/iter reference)
import jax
import jax.numpy as jnp
from jax import lax
from jax.experimental import pallas as pl
from jax.experimental.pallas import tpu as pltpu

N_DEV = 32


def kernel(x, w_mat):
    k_per = x.shape[1]
    m, n = x.shape[0], w_mat.shape[1]
    m_per = m // N_DEV

    def body(x_ref, w_ref, out_ref, partial_ref, acc_ref, send_sems, recv_sems):
        me = lax.axis_index("i")

        partial_ref[...] = jnp.dot(
            x_ref[...], w_ref[...], preferred_element_type=jnp.float32
        )

        rdmas = []
        for k in range(1, N_DEV):
            c = lax.rem(me + k, N_DEV)
            rdma = pltpu.make_async_remote_copy(
                src_ref=partial_ref.at[pl.ds(c * m_per, m_per), :],
                dst_ref=acc_ref.at[me],
                send_sem=send_sems.at[c],
                recv_sem=recv_sems.at[me],
                device_id=(c,),
                device_id_type=pl.DeviceIdType.MESH,
            )
            rdma.start()
            rdmas.append(rdma)

        acc_ref[me] = partial_ref[pl.ds(me * m_per, m_per), :]

        for k in range(1, N_DEV):
            j = lax.rem(me + k, N_DEV)
            recv = pltpu.make_async_remote_copy(
                src_ref=acc_ref.at[j],
                dst_ref=acc_ref.at[j],
                send_sem=send_sems.at[j],
                recv_sem=recv_sems.at[j],
                device_id=(j,),
                device_id_type=pl.DeviceIdType.MESH,
            )
            recv.wait_recv()

        for rdma in rdmas:
            rdma.wait_send()

        out_ref[...] = jnp.sum(acc_ref[...], axis=0)

    return pl.pallas_call(
        body,
        out_shape=jax.ShapeDtypeStruct((m_per, n), jnp.float32),
        in_specs=[
            pl.BlockSpec(memory_space=pltpu.VMEM),
            pl.BlockSpec(memory_space=pltpu.VMEM),
        ],
        out_specs=pl.BlockSpec(memory_space=pltpu.VMEM),
        scratch_shapes=[
            pltpu.VMEM((m, n), jnp.float32),
            pltpu.VMEM((N_DEV, m_per, n), jnp.float32),
            pltpu.SemaphoreType.DMA((N_DEV,)),
            pltpu.SemaphoreType.DMA((N_DEV,)),
        ],
    )(x, w_mat)


# device time: 48930 ns/iter; 1.3343x vs baseline; 1.3343x over previous
import jax
import jax.numpy as jnp
from jax import lax
from jax.experimental import pallas as pl
from jax.experimental.pallas import tpu as pltpu

N_DEV = 32
N_SUB = 16


def kernel(x, w_mat):
    m, n = x.shape[0], w_mat.shape[1]
    m_per = m // N_DEV

    def g(r, x_coord):
        return (r // 4) * 8 + 2 * (r % 4) + ((x_coord + (r % 4)) % 2)

    def body(
        x_ref, w_ref, out_ref,
        partial_ref, p1_buf, stage_buf, p2_buf,
        p1_ssems, p1_rsems, p2_ssems, p2_rsems,
    ):
        me = lax.axis_index("i")
        s_me = me % 8
        z_me = me // 8
        y_me = s_me // 2
        x_me = ((s_me + 1) // 2) % 2
        r_me = z_me * 4 + y_me
        nbr = z_me * 8 + 2 * y_me + ((x_me + 1 + y_me) % 2)

        partial_ref[...] = jnp.dot(
            x_ref[...], w_ref[...], preferred_element_type=jnp.float32
        )

        p1_rdmas = []
        for r in range(N_SUB):
            c_nbr = g(r, 1 - x_me)
            rdma = pltpu.make_async_remote_copy(
                src_ref=partial_ref.at[pl.ds(c_nbr * m_per, m_per), :],
                dst_ref=p1_buf.at[r],
                send_sem=p1_ssems.at[r],
                recv_sem=p1_rsems.at[r],
                device_id=(nbr,),
                device_id_type=pl.DeviceIdType.MESH,
            )
            rdma.start()
            p1_rdmas.append(rdma)

        p2_rdmas = []
        for r in range(N_SUB):
            p1_rdmas[r].wait_recv()
            t = g(r, x_me)
            stage_buf[r] = (
                partial_ref[pl.ds(t * m_per, m_per), :] + p1_buf[r]
            )
            is_me = t == me

            @pl.when(is_me)
            def _():
                p2_buf[r] = stage_buf[r]

            rdma = pltpu.make_async_remote_copy(
                src_ref=stage_buf.at[r],
                dst_ref=p2_buf.at[r_me],
                send_sem=p2_ssems.at[r],
                recv_sem=p2_rsems.at[r_me],
                device_id=(t,),
                device_id_type=pl.DeviceIdType.MESH,
            )

            @pl.when(jnp.logical_not(is_me))
            def _():
                rdma.start()

            p2_rdmas.append((rdma, is_me))

        for r in range(N_SUB):
            recv = pltpu.make_async_remote_copy(
                src_ref=p2_buf.at[r],
                dst_ref=p2_buf.at[r],
                send_sem=p2_ssems.at[r],
                recv_sem=p2_rsems.at[r],
                device_id=(me,),
                device_id_type=pl.DeviceIdType.MESH,
            )

            @pl.when(r != r_me)
            def _():
                recv.wait_recv()

        for rdma in p1_rdmas:
            rdma.wait_send()
        for rdma, is_me in p2_rdmas:

            @pl.when(jnp.logical_not(is_me))
            def _():
                rdma.wait_send()

        out_ref[...] = jnp.sum(p2_buf[...], axis=0)

    return pl.pallas_call(
        body,
        out_shape=jax.ShapeDtypeStruct((m_per, n), jnp.float32),
        in_specs=[
            pl.BlockSpec(memory_space=pltpu.VMEM),
            pl.BlockSpec(memory_space=pltpu.VMEM),
        ],
        out_specs=pl.BlockSpec(memory_space=pltpu.VMEM),
        scratch_shapes=[
            pltpu.VMEM((m, n), jnp.float32),
            pltpu.VMEM((N_SUB, m_per, n), jnp.float32),
            pltpu.VMEM((N_SUB, m_per, n), jnp.float32),
            pltpu.VMEM((N_SUB, m_per, n), jnp.float32),
            pltpu.SemaphoreType.DMA((N_SUB,)),
            pltpu.SemaphoreType.DMA((N_SUB,)),
            pltpu.SemaphoreType.DMA((N_SUB,)),
            pltpu.SemaphoreType.DMA((N_SUB,)),
        ],
    )(x, w_mat)


# device time: 44850 ns/iter; 1.4557x vs baseline; 1.0910x over previous
import jax
import jax.numpy as jnp
from jax import lax
from jax.experimental import pallas as pl
from jax.experimental.pallas import tpu as pltpu

N_DEV = 32
N_SUB = 16


def kernel(x, w_mat):
    m, n = x.shape[0], w_mat.shape[1]
    m_per = m // N_DEV

    def g(r, x_coord):
        return (r // 4) * 8 + 2 * (r % 4) + ((x_coord + (r % 4)) % 2)

    def body(
        x_ref, w_ref, out_ref,
        partial_ref, p1_buf, stage_buf, p2_buf,
        p1_ssems, p1_rsems, p2_ssems, p2_rsems,
    ):
        me = lax.axis_index("i")
        s_me = me % 8
        z_me = me // 8
        y_me = s_me // 2
        x_me = ((s_me + 1) // 2) % 2
        r_me = z_me * 4 + y_me
        nbr = z_me * 8 + 2 * y_me + ((x_me + 1 + y_me) % 2)

        partial_ref[...] = jnp.dot(
            x_ref[...], w_ref[...], preferred_element_type=jnp.float32
        )

        p1_rdmas = []
        for k in range(N_SUB):
            sig = (r_me + 1 + k) % N_SUB
            c_nbr = g(sig, 1 - x_me)
            rdma = pltpu.make_async_remote_copy(
                src_ref=partial_ref.at[pl.ds(c_nbr * m_per, m_per), :],
                dst_ref=p1_buf.at[sig],
                send_sem=p1_ssems.at[sig],
                recv_sem=p1_rsems.at[sig],
                device_id=(nbr,),
                device_id_type=pl.DeviceIdType.MESH,
            )
            rdma.start()
            p1_rdmas.append(rdma)

        p2_rdmas = []
        for k in range(N_SUB - 1):
            sig = (r_me + 1 + k) % N_SUB
            p1_rdmas[k].wait_recv()
            t = g(sig, x_me)
            stage_buf[pl.ds(sig, 1)] = (
                partial_ref[pl.ds(t * m_per, m_per), :] + p1_buf[pl.ds(sig, 1), :, :][0]
            )[None]
            rdma = pltpu.make_async_remote_copy(
                src_ref=stage_buf.at[sig],
                dst_ref=p2_buf.at[r_me],
                send_sem=p2_ssems.at[sig],
                recv_sem=p2_rsems.at[r_me],
                device_id=(t,),
                device_id_type=pl.DeviceIdType.MESH,
            )
            rdma.start()
            p2_rdmas.append(rdma)

        p1_rdmas[N_SUB - 1].wait_recv()
        p2_buf[pl.ds(r_me, 1)] = (
            partial_ref[pl.ds(me * m_per, m_per), :] + p1_buf[pl.ds(r_me, 1), :, :][0]
        )[None]

        for k in range(N_SUB - 1):
            rho = (r_me - 1 - k) % N_SUB
            recv = pltpu.make_async_remote_copy(
                src_ref=p2_buf.at[rho],
                dst_ref=p2_buf.at[rho],
                send_sem=p2_ssems.at[rho],
                recv_sem=p2_rsems.at[rho],
                device_id=(me,),
                device_id_type=pl.DeviceIdType.MESH,
            )
            recv.wait_recv()

        for rdma in p1_rdmas:
            rdma.wait_send()
        for rdma in p2_rdmas:
            rdma.wait_send()

        out_ref[...] = jnp.sum(p2_buf[...], axis=0)

    return pl.pallas_call(
        body,
        out_shape=jax.ShapeDtypeStruct((m_per, n), jnp.float32),
        in_specs=[
            pl.BlockSpec(memory_space=pltpu.VMEM),
            pl.BlockSpec(memory_space=pltpu.VMEM),
        ],
        out_specs=pl.BlockSpec(memory_space=pltpu.VMEM),
        scratch_shapes=[
            pltpu.VMEM((m, n), jnp.float32),
            pltpu.VMEM((N_SUB, m_per, n), jnp.float32),
            pltpu.VMEM((N_SUB, m_per, n), jnp.float32),
            pltpu.VMEM((N_SUB, m_per, n), jnp.float32),
            pltpu.SemaphoreType.DMA((N_SUB,)),
            pltpu.SemaphoreType.DMA((N_SUB,)),
            pltpu.SemaphoreType.DMA((N_SUB,)),
            pltpu.SemaphoreType.DMA((N_SUB,)),
        ],
    )(x, w_mat)


# device time: 40308 ns/iter; 1.6197x vs baseline; 1.1127x over previous
import jax
import jax.numpy as jnp
from jax import lax
from jax.experimental import pallas as pl
from jax.experimental.pallas import tpu as pltpu

N_DEV = 32
N_SUB = 16

OFFS = [8, 7, 9, 6, 10, 5, 11, 4, 12, 3, 13, 2, 14, 1, 15]


def kernel(x, w_mat):
    m, n = x.shape[0], w_mat.shape[1]
    m_per = m // N_DEV

    def g(r, x_coord):
        return (r // 4) * 8 + 2 * (r % 4) + ((x_coord + (r % 4)) % 2)

    def body(
        x_ref, w_ref, out_ref,
        p1_stage, p1_buf, stage_buf, p2_buf,
        p1_ssems, p1_rsems, p2_ssems, p2_rsems,
    ):
        me = lax.axis_index("i")
        s_me = me % 8
        z_me = me // 8
        y_me = s_me // 2
        x_me = ((s_me + 1) // 2) % 2
        r_me = z_me * 4 + y_me
        nbr = z_me * 8 + 2 * y_me + ((x_me + 1 + y_me) % 2)

        barrier_sem = pltpu.get_barrier_semaphore()
        pl.semaphore_signal(
            barrier_sem, inc=1,
            device_id=(nbr,), device_id_type=pl.DeviceIdType.MESH,
        )
        for o in OFFS:
            peer = g((r_me + o) % N_SUB, x_me)
            pl.semaphore_signal(
                barrier_sem, inc=1,
                device_id=(peer,), device_id_type=pl.DeviceIdType.MESH,
            )
        pl.semaphore_wait(barrier_sem, N_SUB)

        p1_rdmas = []
        for k, o in enumerate(OFFS + [N_SUB]):
            sig = (r_me + o) % N_SUB
            c_nbr = g(sig, 1 - x_me)
            p1_stage[pl.ds(sig, 1)] = jnp.dot(
                x_ref[pl.ds(c_nbr * m_per, m_per), :], w_ref[...],
                preferred_element_type=jnp.float32,
            )[None]
            rdma = pltpu.make_async_remote_copy(
                src_ref=p1_stage.at[sig],
                dst_ref=p1_buf.at[sig],
                send_sem=p1_ssems.at[sig],
                recv_sem=p1_rsems.at[sig],
                device_id=(nbr,),
                device_id_type=pl.DeviceIdType.MESH,
            )
            rdma.start()
            p1_rdmas.append(rdma)

        p2_rdmas = []
        for k, o in enumerate(OFFS):
            sig = (r_me + o) % N_SUB
            p1_rdmas[k].wait_recv()
            t = g(sig, x_me)
            stage_buf[pl.ds(sig, 1)] = (
                jnp.dot(
                    x_ref[pl.ds(t * m_per, m_per), :], w_ref[...],
                    preferred_element_type=jnp.float32,
                )
                + p1_buf[pl.ds(sig, 1), :, :][0]
            )[None]
            rdma = pltpu.make_async_remote_copy(
                src_ref=stage_buf.at[sig],
                dst_ref=p2_buf.at[r_me],
                send_sem=p2_ssems.at[sig],
                recv_sem=p2_rsems.at[r_me],
                device_id=(t,),
                device_id_type=pl.DeviceIdType.MESH,
            )
            rdma.start()
            p2_rdmas.append(rdma)

        p1_rdmas[N_SUB - 1].wait_recv()
        p2_buf[pl.ds(r_me, 1)] = (
            jnp.dot(
                x_ref[pl.ds(me * m_per, m_per), :], w_ref[...],
                preferred_element_type=jnp.float32,
            )
            + p1_buf[pl.ds(r_me, 1), :, :][0]
        )[None]

        for o in OFFS:
            rho = (r_me - o) % N_SUB
            recv = pltpu.make_async_remote_copy(
                src_ref=p2_buf.at[rho],
                dst_ref=p2_buf.at[rho],
                send_sem=p2_ssems.at[rho],
                recv_sem=p2_rsems.at[rho],
                device_id=(me,),
                device_id_type=pl.DeviceIdType.MESH,
            )
            recv.wait_recv()

        for rdma in p1_rdmas:
            rdma.wait_send()
        for rdma in p2_rdmas:
            rdma.wait_send()

        out_ref[...] = jnp.sum(p2_buf[...], axis=0)

    return pl.pallas_call(
        body,
        out_shape=jax.ShapeDtypeStruct((m_per, n), jnp.float32),
        in_specs=[
            pl.BlockSpec(memory_space=pltpu.VMEM),
            pl.BlockSpec(memory_space=pltpu.VMEM),
        ],
        out_specs=pl.BlockSpec(memory_space=pltpu.VMEM),
        scratch_shapes=[
            pltpu.VMEM((N_SUB, m_per, n), jnp.float32),
            pltpu.VMEM((N_SUB, m_per, n), jnp.float32),
            pltpu.VMEM((N_SUB, m_per, n), jnp.float32),
            pltpu.VMEM((N_SUB, m_per, n), jnp.float32),
            pltpu.SemaphoreType.DMA((N_SUB,)),
            pltpu.SemaphoreType.DMA((N_SUB,)),
            pltpu.SemaphoreType.DMA((N_SUB,)),
            pltpu.SemaphoreType.DMA((N_SUB,)),
        ],
        compiler_params=pltpu.CompilerParams(collective_id=0),
    )(x, w_mat)
